# Initial kernel scaffold; baseline (speedup 1.0000x reference)
import jax
import jax.numpy as jnp
from jax.experimental import pallas as pl


def kernel(src, feature_map, W, b):
    raise NotImplementedError("write your pallas kernel here")


# speedup vs baseline: 1.9832x; 1.9832x over previous
import functools

import jax
import jax.numpy as jnp
from jax import lax
from jax.experimental import pallas as pl
from jax.experimental.pallas import tpu as pltpu
from jax.experimental.pallas import tpu_sc as plsc

D = 128
TROWS = 128
KPAD = 8
C = 128
NC = 2
NS = 16
NW = NC * NS


def _proj_body(fm_ref, w_ref, b_ref, o_ref):
    o_ref[...] = (
        jnp.dot(fm_ref[...], w_ref[...], preferred_element_type=jnp.float32)
        + b_ref[...]
    )


def _build_table(fm_pad, w_pad, b_row):
    return pl.pallas_call(
        _proj_body,
        out_shape=jax.ShapeDtypeStruct((TROWS, D), jnp.float32),
    )(fm_pad, w_pad, b_row)


def _make_gather(n_rows):
    nchunk = n_rows // C
    nk = -(-nchunk // NW)

    mesh = plsc.VectorSubcoreMesh(core_axis_name="c", subcore_axis_name="s")

    @functools.partial(
        pl.kernel,
        mesh=mesh,
        out_type=jax.ShapeDtypeStruct((n_rows, D), jnp.float32),
        scratch_types=[
            pltpu.VMEM((C,), jnp.int32),
            pltpu.VMEM((C, D), jnp.float32),
            pltpu.SemaphoreType.DMA,
        ],
    )
    def gather(table_hbm, idx_hbm, out_hbm, idx_v, rows_v, sem):
        wid = lax.axis_index("s") * NC + lax.axis_index("c")

        def body(k, carry):
            g = k * NW + wid

            @pl.when(g < nchunk)
            def _():
                base = g * C
                pltpu.sync_copy(idx_hbm.at[pl.ds(base, C)], idx_v)
                pltpu.async_copy(table_hbm.at[idx_v], rows_v, sem).wait()
                pltpu.sync_copy(rows_v, out_hbm.at[pl.ds(base, C)])

            return carry

        lax.fori_loop(0, nk, body, 0)

    return gather


def kernel(src, feature_map, W, b):
    fm_pad = jnp.zeros((TROWS, KPAD), jnp.float32).at[:119, :3].set(feature_map)
    w_pad = jnp.zeros((KPAD, D), jnp.float32).at[:3].set(W)
    table = _build_table(fm_pad, w_pad, b.reshape(1, D).astype(jnp.float32))
    idx = src.astype(jnp.int32)
    return _make_gather(src.shape[0])(table, idx)
